# baseline (speedup 1.0000x reference)
import jax
import jax.numpy as jnp
from jax import lax
from jax.experimental import pallas as pl
from jax.experimental.pallas import tpu as pltpu

D_MODEL = 2048
NUM_EXPERTS = 16
NUM_SELECTED = 2
CAPACITY_FACTOR = 1.25
Z_LOSS_COEF = 0.01

TOKEN_BLOCK = 512
NEG_HUGE = -3.0e38


def _router_block(w_ref, x_ref, it_ref, wt_ref, cnt_ref, lse_ref, ent_ref):
    step = pl.program_id(0)

    logits = lax.dot_general(
        w_ref[...], x_ref[...],
        dimension_numbers=(((1,), (1,)), ((), ())),
        preferred_element_type=jnp.float32)

    m1 = jnp.max(logits, axis=0, keepdims=True)
    e = jnp.exp(logits - m1)
    s = jnp.sum(e, axis=0, keepdims=True)
    lse = m1 + jnp.log(s)
    sel = jnp.sum(e * logits, axis=0, keepdims=True)
    ent = lse - sel / s
    block_lse = jnp.sum(lse)[None, None]
    block_ent = jnp.sum(ent)[None, None]

    rows = lax.broadcasted_iota(jnp.int32, logits.shape, 0)
    big = jnp.int32(NUM_EXPERTS)
    i1 = jnp.min(jnp.where(logits == m1, rows, big), axis=0,
                 keepdims=True)
    vm = jnp.where(rows == i1, NEG_HUGE, logits)
    m2 = jnp.max(vm, axis=0, keepdims=True)
    i2 = jnp.min(jnp.where(vm == m2, rows, big), axis=0, keepdims=True)

    rs = 1.0 / s
    p1 = rs
    p2 = jnp.exp(m2 - m1) * rs
    rden = 1.0 / (p1 + p2 + 1e-8)
    w1 = p1 * rden
    w2 = p2 * rden

    it_ref[...] = jnp.concatenate([i1, i2], axis=0)
    wt_ref[...] = jnp.concatenate([w1, w2], axis=0)

    hit = (jnp.where(rows == i1, 1.0, 0.0) +
           jnp.where(rows == i2, 1.0, 0.0))
    block_cnt = jnp.sum(hit, axis=1, keepdims=True)

    @pl.when(step == 0)
    def _init():
        lse_ref[...] = block_lse
        ent_ref[...] = block_ent
        cnt_ref[...] = block_cnt

    @pl.when(step != 0)
    def _acc():
        lse_ref[...] += block_lse
        ent_ref[...] += block_ent
        cnt_ref[...] += block_cnt


@jax.jit
def _router(gate_weight, hidden_flat):
    n_tokens = hidden_flat.shape[0]
    grid = (n_tokens // TOKEN_BLOCK,)
    return pl.pallas_call(
        _router_block,
        grid=grid,
        in_specs=[
            pl.BlockSpec((NUM_EXPERTS, D_MODEL), lambda i: (0, 0)),
            pl.BlockSpec((TOKEN_BLOCK, D_MODEL), lambda i: (i, 0)),
        ],
        out_specs=(
            pl.BlockSpec((NUM_SELECTED, TOKEN_BLOCK), lambda i: (0, i)),
            pl.BlockSpec((NUM_SELECTED, TOKEN_BLOCK), lambda i: (0, i)),
            pl.BlockSpec((NUM_EXPERTS, 1), lambda i: (0, 0)),
            pl.BlockSpec((1, 1), lambda i: (0, 0)),
            pl.BlockSpec((1, 1), lambda i: (0, 0)),
        ),
        out_shape=(
            jax.ShapeDtypeStruct((NUM_SELECTED, n_tokens), jnp.int32),
            jax.ShapeDtypeStruct((NUM_SELECTED, n_tokens), jnp.float32),
            jax.ShapeDtypeStruct((NUM_EXPERTS, 1), jnp.float32),
            jax.ShapeDtypeStruct((1, 1), jnp.float32),
            jax.ShapeDtypeStruct((1, 1), jnp.float32),
        ),
        compiler_params=pltpu.CompilerParams(
            dimension_semantics=("arbitrary",),
        ),
    )(gate_weight, hidden_flat)


def kernel(hidden_states, gate_weight):
    batch_size, seq_len, d_model = hidden_states.shape
    num_tokens = batch_size * seq_len
    hidden_flat = hidden_states.reshape(num_tokens, d_model)

    it, wt, cnt, lse_sum, ent_sum = _router(gate_weight, hidden_flat)

    expert_counts = cnt[:, 0]
    capacity = int(CAPACITY_FACTOR * num_tokens / NUM_EXPERTS * NUM_SELECTED)
    expert_overflow = jnp.sum(jnp.maximum(expert_counts - capacity, 0.0))
    capacity_overflow_pct = expert_overflow / num_tokens * 100.0
    z_loss = lse_sum[0, 0] / num_tokens * Z_LOSS_COEF
    gate_entropy = ent_sum[0, 0] / num_tokens
    expert_load_normalized = expert_counts / jnp.sum(expert_counts)
    ideal_load = 1.0 / NUM_EXPERTS
    expert_load_variance = jnp.mean((expert_load_normalized - ideal_load) ** 2)

    expert_indices = it.T.reshape(batch_size, seq_len, NUM_SELECTED)
    expert_weights = wt.T.reshape(batch_size, seq_len, NUM_SELECTED)
    routing_confidence = wt[0]
    return (expert_indices, expert_weights, expert_counts,
            capacity_overflow_pct, z_loss, gate_entropy,
            expert_load_variance, routing_confidence)


# speedup vs baseline: 1.1451x; 1.1451x over previous
import jax
import jax.numpy as jnp
from jax import lax
from jax.experimental import pallas as pl
from jax.experimental.pallas import tpu as pltpu

D_MODEL = 2048
NUM_EXPERTS = 16
NUM_SELECTED = 2
CAPACITY_FACTOR = 1.25
Z_LOSS_COEF = 0.01

TOKEN_BLOCK = 1024
NEG_HUGE = -3.0e38


def _router_block(w_ref, x0_ref, x1_ref, it_ref, wt_ref, cnt_ref,
                  lse_ref, ent_ref):
    step = pl.program_id(0)

    l0 = lax.dot_general(
        w_ref[...], x0_ref[...],
        dimension_numbers=(((1,), (1,)), ((), ())),
        preferred_element_type=jnp.float32)
    l1 = lax.dot_general(
        w_ref[...], x1_ref[...],
        dimension_numbers=(((1,), (1,)), ((), ())),
        preferred_element_type=jnp.float32)
    logits = jnp.concatenate([l0, l1], axis=1)

    m1 = jnp.max(logits, axis=0, keepdims=True)
    e = jnp.exp(logits - m1)
    s = jnp.sum(e, axis=0, keepdims=True)
    lse = m1 + jnp.log(s)
    sel = jnp.sum(e * logits, axis=0, keepdims=True)
    ent = lse - sel / s
    block_lse = jnp.sum(lse)[None, None]
    block_ent = jnp.sum(ent)[None, None]

    rows = lax.broadcasted_iota(jnp.int32, logits.shape, 0)
    big = jnp.int32(NUM_EXPERTS)
    i1 = jnp.min(jnp.where(logits == m1, rows, big), axis=0,
                 keepdims=True)
    vm = jnp.where(rows == i1, NEG_HUGE, logits)
    m2 = jnp.max(vm, axis=0, keepdims=True)
    i2 = jnp.min(jnp.where(vm == m2, rows, big), axis=0, keepdims=True)

    rs = 1.0 / s
    p1 = rs
    p2 = jnp.exp(m2 - m1) * rs
    rden = 1.0 / (p1 + p2 + 1e-8)
    w1 = p1 * rden
    w2 = p2 * rden

    it_ref[...] = jnp.concatenate([i1, i2], axis=0)
    wt_ref[...] = jnp.concatenate([w1, w2], axis=0)

    hit = (jnp.where(rows == i1, 1.0, 0.0) +
           jnp.where(rows == i2, 1.0, 0.0))
    block_cnt = jnp.sum(hit, axis=1, keepdims=True)

    @pl.when(step == 0)
    def _init():
        lse_ref[...] = block_lse
        ent_ref[...] = block_ent
        cnt_ref[...] = block_cnt

    @pl.when(step != 0)
    def _acc():
        lse_ref[...] += block_lse
        ent_ref[...] += block_ent
        cnt_ref[...] += block_cnt


@jax.jit
def _router(gate_weight, hidden_flat):
    n_tokens = hidden_flat.shape[0]
    grid = (n_tokens // TOKEN_BLOCK,)
    return pl.pallas_call(
        _router_block,
        grid=grid,
        in_specs=[
            pl.BlockSpec((NUM_EXPERTS, D_MODEL), lambda i: (0, 0)),
            pl.BlockSpec((TOKEN_BLOCK // 2, D_MODEL), lambda i: (2 * i, 0)),
            pl.BlockSpec((TOKEN_BLOCK // 2, D_MODEL),
                         lambda i: (2 * i + 1, 0)),
        ],
        out_specs=(
            pl.BlockSpec((NUM_SELECTED, TOKEN_BLOCK), lambda i: (0, i)),
            pl.BlockSpec((NUM_SELECTED, TOKEN_BLOCK), lambda i: (0, i)),
            pl.BlockSpec((NUM_EXPERTS, 1), lambda i: (0, 0)),
            pl.BlockSpec((1, 1), lambda i: (0, 0)),
            pl.BlockSpec((1, 1), lambda i: (0, 0)),
        ),
        out_shape=(
            jax.ShapeDtypeStruct((NUM_SELECTED, n_tokens), jnp.int32),
            jax.ShapeDtypeStruct((NUM_SELECTED, n_tokens), jnp.float32),
            jax.ShapeDtypeStruct((NUM_EXPERTS, 1), jnp.float32),
            jax.ShapeDtypeStruct((1, 1), jnp.float32),
            jax.ShapeDtypeStruct((1, 1), jnp.float32),
        ),
        compiler_params=pltpu.CompilerParams(
            dimension_semantics=("arbitrary",),
        ),
    )(gate_weight, hidden_flat, hidden_flat)


def kernel(hidden_states, gate_weight):
    batch_size, seq_len, d_model = hidden_states.shape
    num_tokens = batch_size * seq_len
    hidden_flat = hidden_states.reshape(num_tokens, d_model)

    it, wt, cnt, lse_sum, ent_sum = _router(gate_weight, hidden_flat)

    expert_counts = cnt[:, 0]
    capacity = int(CAPACITY_FACTOR * num_tokens / NUM_EXPERTS * NUM_SELECTED)
    expert_overflow = jnp.sum(jnp.maximum(expert_counts - capacity, 0.0))
    capacity_overflow_pct = expert_overflow / num_tokens * 100.0
    z_loss = lse_sum[0, 0] / num_tokens * Z_LOSS_COEF
    gate_entropy = ent_sum[0, 0] / num_tokens
    expert_load_normalized = expert_counts / jnp.sum(expert_counts)
    ideal_load = 1.0 / NUM_EXPERTS
    expert_load_variance = jnp.mean((expert_load_normalized - ideal_load) ** 2)

    expert_indices = it.T.reshape(batch_size, seq_len, NUM_SELECTED)
    expert_weights = wt.T.reshape(batch_size, seq_len, NUM_SELECTED)
    routing_confidence = wt[0]
    return (expert_indices, expert_weights, expert_counts,
            capacity_overflow_pct, z_loss, gate_entropy,
            expert_load_variance, routing_confidence)


# speedup vs baseline: 1.1681x; 1.0201x over previous
import jax
import jax.numpy as jnp
from jax import lax
from jax.experimental import pallas as pl
from jax.experimental.pallas import tpu as pltpu

D_MODEL = 2048
NUM_EXPERTS = 16
NUM_SELECTED = 2
CAPACITY_FACTOR = 1.25
Z_LOSS_COEF = 0.01

TOKEN_BLOCK = 1024
NEG_HUGE = -3.0e38


def _router_block(w_ref, x_ref, it_ref, wt_ref, cnt_ref, lse_ref, ent_ref):
    step = pl.program_id(0)

    logits = lax.dot_general(
        w_ref[...], x_ref[...],
        dimension_numbers=(((1,), (1,)), ((), ())),
        preferred_element_type=jnp.float32)

    m1 = jnp.max(logits, axis=0, keepdims=True)
    e = jnp.exp(logits - m1)
    s = jnp.sum(e, axis=0, keepdims=True)
    lse = m1 + jnp.log(s)
    sel = jnp.sum(e * logits, axis=0, keepdims=True)
    ent = lse - sel / s
    block_lse = jnp.sum(lse)[None, None]
    block_ent = jnp.sum(ent)[None, None]

    rows = lax.broadcasted_iota(jnp.int32, logits.shape, 0)
    big = jnp.int32(NUM_EXPERTS)
    i1 = jnp.min(jnp.where(logits == m1, rows, big), axis=0,
                 keepdims=True)
    vm = jnp.where(rows == i1, NEG_HUGE, logits)
    m2 = jnp.max(vm, axis=0, keepdims=True)
    i2 = jnp.min(jnp.where(vm == m2, rows, big), axis=0, keepdims=True)

    rs = 1.0 / s
    p1 = rs
    p2 = jnp.exp(m2 - m1) * rs
    rden = 1.0 / (p1 + p2 + 1e-8)
    w1 = p1 * rden
    w2 = p2 * rden

    it_ref[...] = jnp.concatenate([i1, i2], axis=0)
    wt_ref[...] = jnp.concatenate([w1, w2], axis=0)

    hit = (jnp.where(rows == i1, 1.0, 0.0) +
           jnp.where(rows == i2, 1.0, 0.0))
    block_cnt = jnp.sum(hit, axis=1, keepdims=True)

    @pl.when(step == 0)
    def _init():
        lse_ref[...] = block_lse
        ent_ref[...] = block_ent
        cnt_ref[...] = block_cnt

    @pl.when(step != 0)
    def _acc():
        lse_ref[...] += block_lse
        ent_ref[...] += block_ent
        cnt_ref[...] += block_cnt


@jax.jit
def _router(gate_weight, hidden_flat):
    n_tokens = hidden_flat.shape[0]
    grid = (n_tokens // TOKEN_BLOCK,)
    return pl.pallas_call(
        _router_block,
        grid=grid,
        in_specs=[
            pl.BlockSpec((NUM_EXPERTS, D_MODEL), lambda i: (0, 0)),
            pl.BlockSpec((TOKEN_BLOCK, D_MODEL), lambda i: (i, 0)),
        ],
        out_specs=(
            pl.BlockSpec((NUM_SELECTED, TOKEN_BLOCK), lambda i: (0, i)),
            pl.BlockSpec((NUM_SELECTED, TOKEN_BLOCK), lambda i: (0, i)),
            pl.BlockSpec((NUM_EXPERTS, 1), lambda i: (0, 0)),
            pl.BlockSpec((1, 1), lambda i: (0, 0)),
            pl.BlockSpec((1, 1), lambda i: (0, 0)),
        ),
        out_shape=(
            jax.ShapeDtypeStruct((NUM_SELECTED, n_tokens), jnp.int32),
            jax.ShapeDtypeStruct((NUM_SELECTED, n_tokens), jnp.float32),
            jax.ShapeDtypeStruct((NUM_EXPERTS, 1), jnp.float32),
            jax.ShapeDtypeStruct((1, 1), jnp.float32),
            jax.ShapeDtypeStruct((1, 1), jnp.float32),
        ),
        compiler_params=pltpu.CompilerParams(
            dimension_semantics=("arbitrary",),
        ),
    )(gate_weight, hidden_flat)


def kernel(hidden_states, gate_weight):
    batch_size, seq_len, d_model = hidden_states.shape
    num_tokens = batch_size * seq_len
    hidden_flat = hidden_states.reshape(num_tokens, d_model)

    it, wt, cnt, lse_sum, ent_sum = _router(gate_weight, hidden_flat)

    expert_counts = cnt[:, 0]
    capacity = int(CAPACITY_FACTOR * num_tokens / NUM_EXPERTS * NUM_SELECTED)
    expert_overflow = jnp.sum(jnp.maximum(expert_counts - capacity, 0.0))
    capacity_overflow_pct = expert_overflow / num_tokens * 100.0
    z_loss = lse_sum[0, 0] / num_tokens * Z_LOSS_COEF
    gate_entropy = ent_sum[0, 0] / num_tokens
    expert_load_normalized = expert_counts / jnp.sum(expert_counts)
    ideal_load = 1.0 / NUM_EXPERTS
    expert_load_variance = jnp.mean((expert_load_normalized - ideal_load) ** 2)

    expert_indices = it.T.reshape(batch_size, seq_len, NUM_SELECTED)
    expert_weights = wt.T.reshape(batch_size, seq_len, NUM_SELECTED)
    routing_confidence = wt[0]
    return (expert_indices, expert_weights, expert_counts,
            capacity_overflow_pct, z_loss, gate_entropy,
            expert_load_variance, routing_confidence)
